# Initial kernel scaffold; baseline (speedup 1.0000x reference)
import jax
import jax.numpy as jnp
from jax.experimental import pallas as pl


def kernel(img, index_flat_inv):
    raise NotImplementedError("write your pallas kernel here")


# speedup vs baseline: 4.2787x; 4.2787x over previous
import functools

import jax
import jax.numpy as jnp
from jax import lax
from jax.experimental import pallas as pl
from jax.experimental.pallas import tpu as pltpu
from jax.experimental.pallas import tpu_sc as plsc


def kernel(img, index_flat_inv):
    B, T, C = img.shape
    idx = index_flat_inv.astype(jnp.int32)

    info = plsc.get_sparse_core_info()
    NC, NS = info.num_cores, info.num_subcores
    NW = NC * NS
    assert B == NW

    CH = 128
    n_ch = T // CH
    idx2 = idx.reshape(n_ch, CH)

    mesh = plsc.VectorSubcoreMesh(core_axis_name="c", subcore_axis_name="s")

    @functools.partial(
        pl.kernel,
        mesh=mesh,
        out_type=jax.ShapeDtypeStruct((B, T, C), jnp.float32),
        scratch_types=[
            pltpu.VMEM((n_ch, CH), jnp.int32),
            pltpu.VMEM((CH, C), jnp.float32),
            pltpu.SemaphoreType.DMA,
        ],
    )
    def k(img_hbm, idx_hbm, out_hbm, perm_v, rows_v, sem):
        wid = lax.axis_index("s") * NC + lax.axis_index("c")
        pltpu.sync_copy(idx_hbm, perm_v)
        for j in range(n_ch):
            pltpu.sync_copy(img_hbm.at[wid, pl.ds(j * CH, CH)], rows_v)
            pltpu.async_copy(rows_v, out_hbm.at[wid].at[perm_v.at[j]], sem).wait()

    return k(img, idx2)


# speedup vs baseline: 4.3279x; 1.0115x over previous
import functools

import jax
import jax.numpy as jnp
from jax import lax
from jax.experimental import pallas as pl
from jax.experimental.pallas import tpu as pltpu
from jax.experimental.pallas import tpu_sc as plsc


def kernel(img, index_flat_inv):
    B, T, C = img.shape
    idx = index_flat_inv.astype(jnp.int32)

    info = plsc.get_sparse_core_info()
    NC, NS = info.num_cores, info.num_subcores
    NW = NC * NS
    assert B == NW

    CH = 64
    n_ch = T // CH
    idx2 = idx.reshape(n_ch, CH)

    mesh = plsc.VectorSubcoreMesh(core_axis_name="c", subcore_axis_name="s")

    @functools.partial(
        pl.kernel,
        mesh=mesh,
        out_type=jax.ShapeDtypeStruct((B, T, C), jnp.float32),
        scratch_types=[
            pltpu.VMEM((n_ch, CH), jnp.int32),
            pltpu.VMEM((CH, C), jnp.float32),
            pltpu.VMEM((CH, C), jnp.float32),
            pltpu.SemaphoreType.DMA,
            pltpu.SemaphoreType.DMA,
            pltpu.SemaphoreType.DMA,
            pltpu.SemaphoreType.DMA,
        ],
    )
    def k(img_hbm, idx_hbm, out_hbm, perm_v, rows0, rows1, rs0, rs1, ws0, ws1):
        wid = lax.axis_index("s") * NC + lax.axis_index("c")
        bufs, rsems, wsems = [rows0, rows1], [rs0, rs1], [ws0, ws1]
        pltpu.sync_copy(idx_hbm, perm_v)
        rd = [None, None]
        wr = [None, None]
        rd[0] = pltpu.async_copy(img_hbm.at[wid, pl.ds(0, CH)], bufs[0], rsems[0])
        for j in range(n_ch):
            cur, nxt = j % 2, (j + 1) % 2
            if j + 1 < n_ch:
                if wr[nxt] is not None:
                    wr[nxt].wait()
                rd[nxt] = pltpu.async_copy(
                    img_hbm.at[wid, pl.ds((j + 1) * CH, CH)], bufs[nxt], rsems[nxt])
            rd[cur].wait()
            wr[cur] = pltpu.async_copy(
                bufs[cur], out_hbm.at[wid].at[perm_v.at[j]], wsems[cur])
        for w in wr:
            if w is not None:
                w.wait()

    return k(img, idx2)
